# baseline (speedup 1.0000x reference)
import jax
import jax.numpy as jnp
from jax.experimental import pallas as pl
from jax.experimental.pallas import tpu as pltpu

M = 32
K = 256
D = 24
EMB = M * D


def _pq_kernel(x_ref, cb_ref, cbt_ref, quantT0_ref, quantT1_ref, codesT_ref):
    x = x_ref[:]
    bt = x.shape[0]
    codes_rows = []
    quant_rows = []
    for m in range(M):
        cb_m = cb_ref[m]
        cbt_m = cbt_ref[m]
        x_m = x[:, m * D:(m + 1) * D]
        scoresT = jax.lax.dot_general(cb_m, x_m, (((1,), (1,)), ((), ())))
        codes_m = jnp.argmax(scoresT, axis=0).astype(jnp.int32)
        idx = jnp.broadcast_to(codes_m[None, :], (D, bt))
        q0 = jnp.take_along_axis(cbt_m[:, :128], jnp.minimum(idx, 127), axis=1)
        q1 = jnp.take_along_axis(cbt_m[:, 128:], jnp.maximum(idx - 128, 0),
                                 axis=1)
        quant_rows.append(jnp.where(idx < 128, q0, q1))
        codes_rows.append(codes_m[None, :])
    quantT0_ref[:] = jnp.concatenate(quant_rows[:16], axis=0)
    quantT1_ref[:] = jnp.concatenate(quant_rows[16:], axis=0)
    codesT_ref[:] = jnp.concatenate(codes_rows, axis=0)


def kernel(x, codebook):
    B = x.shape[0]
    BT = 512
    grid = (B // BT,)
    cbt = codebook.transpose(0, 2, 1)
    quantT0, quantT1, codesT = pl.pallas_call(
        _pq_kernel,
        grid=grid,
        in_specs=[pl.BlockSpec((BT, EMB), lambda i: (i, 0)),
                  pl.BlockSpec((M, K, D), lambda i: (0, 0, 0)),
                  pl.BlockSpec((M, D, K), lambda i: (0, 0, 0))],
        out_specs=[pl.BlockSpec((EMB // 2, BT), lambda i: (0, i)),
                   pl.BlockSpec((EMB // 2, BT), lambda i: (0, i)),
                   pl.BlockSpec((M, BT), lambda i: (0, i))],
        out_shape=(jax.ShapeDtypeStruct((EMB // 2, B), jnp.float32),
                   jax.ShapeDtypeStruct((EMB // 2, B), jnp.int32).update(dtype=jnp.float32),
                   jax.ShapeDtypeStruct((M, B), jnp.int32)),
        compiler_params=pltpu.CompilerParams(
            dimension_semantics=("parallel",)),
    )(x, codebook, cbt)
    quant = jnp.concatenate([quantT0.T, quantT1.T], axis=1)
    return quant, codesT.T


# speedup vs baseline: 1.0240x; 1.0240x over previous
import jax
import jax.numpy as jnp
from jax.experimental import pallas as pl
from jax.experimental.pallas import tpu as pltpu

M = 32
K = 256
D = 24
EMB = M * D


def _pq_kernel(x_ref, cb_ref, cbt_ref, quantT_ref, codesT_ref):
    x = x_ref[:]
    bt = x.shape[0]
    codes_rows = []
    quant_rows = []
    for m in range(M):
        cb_m = cb_ref[m]
        cbt_m = cbt_ref[m]
        x_m = x[:, m * D:(m + 1) * D]
        scoresT = jax.lax.dot_general(cb_m, x_m, (((1,), (1,)), ((), ())))
        codes_m = jnp.argmax(scoresT, axis=0).astype(jnp.int32)
        idx = jnp.broadcast_to(codes_m[None, :], (D, bt))
        q0 = jnp.take_along_axis(cbt_m[:, :128], jnp.minimum(idx, 127), axis=1)
        q1 = jnp.take_along_axis(cbt_m[:, 128:], jnp.maximum(idx - 128, 0),
                                 axis=1)
        quant_rows.append(jnp.where(idx < 128, q0, q1))
        codes_rows.append(codes_m[None, :])
    quantT_ref[:] = jnp.concatenate(quant_rows, axis=0)
    codesT_ref[:] = jnp.concatenate(codes_rows, axis=0)


def kernel(x, codebook):
    B = x.shape[0]
    BT = 1024
    grid = (B // BT,)
    cbt = codebook.transpose(0, 2, 1)
    quantT, codesT = pl.pallas_call(
        _pq_kernel,
        grid=grid,
        in_specs=[pl.BlockSpec((BT, EMB), lambda i: (i, 0)),
                  pl.BlockSpec((M, K, D), lambda i: (0, 0, 0)),
                  pl.BlockSpec((M, D, K), lambda i: (0, 0, 0))],
        out_specs=[pl.BlockSpec((EMB, BT), lambda i: (0, i)),
                   pl.BlockSpec((M, BT), lambda i: (0, i))],
        out_shape=(jax.ShapeDtypeStruct((EMB, B), jnp.float32),
                   jax.ShapeDtypeStruct((M, B), jnp.int32)),
        compiler_params=pltpu.CompilerParams(
            dimension_semantics=("parallel",)),
    )(x, codebook, cbt)
    return quantT.T, codesT.T


# speedup vs baseline: 1.1765x; 1.1490x over previous
import jax
import jax.numpy as jnp
from jax.experimental import pallas as pl
from jax.experimental.pallas import tpu as pltpu

M = 32
K = 256
D = 24
EMB = M * D


def _pq_kernel(x_ref, cb_ref, cbt_ref, quantT_ref, codesT_ref):
    x = x_ref[:]
    bt = x.shape[0]
    codes_rows = []
    quant_rows = []
    for m in range(M):
        cb_m = cb_ref[m]
        cbt_m = cbt_ref[m]
        x_m = x[:, m * D:(m + 1) * D]
        scoresT = jax.lax.dot_general(cb_m, x_m, (((1,), (1,)), ((), ())))
        codes_m = jnp.argmax(scoresT, axis=0).astype(jnp.int32)
        idx = jnp.broadcast_to(codes_m[None, :], (D, bt))
        q0 = jnp.take_along_axis(cbt_m[:, :128], jnp.minimum(idx, 127), axis=1)
        q1 = jnp.take_along_axis(cbt_m[:, 128:], jnp.maximum(idx - 128, 0),
                                 axis=1)
        quant_rows.append(jnp.where(idx < 128, q0, q1))
        codes_rows.append(codes_m[None, :])
    quantT_ref[:] = jnp.concatenate(quant_rows, axis=0)
    codesT_ref[:] = jnp.concatenate(codes_rows, axis=0)


def kernel(x, codebook):
    B = x.shape[0]
    BT = 256
    grid = (B // BT,)
    cbt = codebook.transpose(0, 2, 1)
    quantT, codesT = pl.pallas_call(
        _pq_kernel,
        grid=grid,
        in_specs=[pl.BlockSpec((BT, EMB), lambda i: (i, 0)),
                  pl.BlockSpec((M, K, D), lambda i: (0, 0, 0)),
                  pl.BlockSpec((M, D, K), lambda i: (0, 0, 0))],
        out_specs=[pl.BlockSpec((EMB, BT), lambda i: (0, i)),
                   pl.BlockSpec((M, BT), lambda i: (0, i))],
        out_shape=(jax.ShapeDtypeStruct((EMB, B), jnp.float32),
                   jax.ShapeDtypeStruct((M, B), jnp.int32)),
        compiler_params=pltpu.CompilerParams(
            dimension_semantics=("parallel",)),
    )(x, codebook, cbt)
    return quantT.T, codesT.T
